# Initial kernel scaffold; baseline (speedup 1.0000x reference)
import jax
import jax.numpy as jnp
from jax.experimental import pallas as pl


def kernel(input_ids, table):
    raise NotImplementedError("write your pallas kernel here")


# speedup vs baseline: 1.6351x; 1.6351x over previous
import functools

import jax
import jax.numpy as jnp
from jax import lax
from jax.experimental import pallas as pl
from jax.experimental.pallas import tpu as pltpu
from jax.experimental.pallas import tpu_sc as plsc

HIDDEN = 2048
NUM_WORKERS = 32
CHUNK = 32


def _emb_body(idx_hbm, table_hbm, out_hbm, idx_v, rows_v, sem, *, bpw, n_chunks):
    wid = lax.axis_index("s") * 2 + lax.axis_index("c")
    base = wid * bpw
    pltpu.sync_copy(idx_hbm.at[pl.ds(base, bpw)], idx_v)

    def body(g, carry):
        off = g * CHUNK
        pltpu.async_copy(
            table_hbm.at[idx_v.at[pl.ds(off, CHUNK)]], rows_v, sem
        ).wait()
        pltpu.sync_copy(rows_v, out_hbm.at[pl.ds(base + off, CHUNK)])
        return carry

    lax.fori_loop(0, n_chunks, body, 0)


def kernel(input_ids, table):
    b, t = input_ids.shape
    n = b * t
    idx = input_ids.reshape(n).astype(jnp.int32)
    bpw = n // NUM_WORKERS
    n_chunks = bpw // CHUNK

    mesh = plsc.VectorSubcoreMesh(core_axis_name="c", subcore_axis_name="s")
    emb = pl.kernel(
        functools.partial(_emb_body, bpw=bpw, n_chunks=n_chunks),
        mesh=mesh,
        out_type=jax.ShapeDtypeStruct((n, HIDDEN), jnp.float32),
        scratch_types=[
            pltpu.VMEM((bpw,), jnp.int32),
            pltpu.VMEM((CHUNK, HIDDEN), jnp.float32),
            pltpu.SemaphoreType.DMA,
        ],
    )
    out = emb(idx, table)
    return out.reshape(b, t, HIDDEN)


# speedup vs baseline: 1.7773x; 1.0869x over previous
import functools

import jax
import jax.numpy as jnp
from jax import lax
from jax.experimental import pallas as pl
from jax.experimental.pallas import tpu as pltpu
from jax.experimental.pallas import tpu_sc as plsc

HIDDEN = 2048
NUM_WORKERS = 32
CHUNK = 16


def _emb_body(idx_hbm, table_hbm, out_hbm, idx_v, buf0, buf1, sem0, sem1,
              *, bpw, n_chunks):
    wid = lax.axis_index("s") * 2 + lax.axis_index("c")
    base = wid * bpw
    pltpu.sync_copy(idx_hbm.at[pl.ds(base, bpw)], idx_v)

    def gather(g, buf, sem):
        return pltpu.make_async_copy(
            table_hbm.at[idx_v.at[pl.ds(g * CHUNK, CHUNK)]], buf, sem
        )

    gather(0, buf0, sem0).start()

    def body(k, carry):
        g0 = 2 * k
        gather(g0 + 1, buf1, sem1).start()
        gather(g0, buf0, sem0).wait()
        pltpu.sync_copy(buf0, out_hbm.at[pl.ds(base + g0 * CHUNK, CHUNK)])

        @pl.when(g0 + 2 < n_chunks)
        def _():
            gather(g0 + 2, buf0, sem0).start()

        gather(g0 + 1, buf1, sem1).wait()
        pltpu.sync_copy(buf1, out_hbm.at[pl.ds(base + (g0 + 1) * CHUNK, CHUNK)])
        return carry

    lax.fori_loop(0, n_chunks // 2, body, 0)


def kernel(input_ids, table):
    b, t = input_ids.shape
    n = b * t
    idx = input_ids.reshape(n).astype(jnp.int32)
    bpw = n // NUM_WORKERS
    n_chunks = bpw // CHUNK

    mesh = plsc.VectorSubcoreMesh(core_axis_name="c", subcore_axis_name="s")
    emb = pl.kernel(
        functools.partial(_emb_body, bpw=bpw, n_chunks=n_chunks),
        mesh=mesh,
        out_type=jax.ShapeDtypeStruct((n, HIDDEN), jnp.float32),
        scratch_types=[
            pltpu.VMEM((bpw,), jnp.int32),
            pltpu.VMEM((CHUNK, HIDDEN), jnp.float32),
            pltpu.VMEM((CHUNK, HIDDEN), jnp.float32),
            pltpu.SemaphoreType.DMA,
            pltpu.SemaphoreType.DMA,
        ],
    )
    out = emb(idx, table)
    return out.reshape(b, t, HIDDEN)
